# baseline (speedup 1.0000x reference)
import functools
import jax
import jax.numpy as jnp
from jax import lax
from jax.experimental import pallas as pl
from jax.experimental.pallas import tpu as pltpu
from jax.experimental.pallas import tpu_sc as plsc

DIM_POS = 64
DIM_TOK = 128
DIM_OUT = DIM_POS + DIM_TOK

_NC = 2
_NS = 16
_NW = _NC * _NS
_L = 16


def _make_kernel(batch, seq):
    assert batch % (_NW * _L * 8) == 0
    bpw = batch // _NW
    mesh = plsc.VectorSubcoreMesh(core_axis_name="c", subcore_axis_name="s")

    @functools.partial(
        pl.kernel,
        out_type=jax.ShapeDtypeStruct((seq, DIM_OUT, batch), jnp.float32),
        mesh=mesh,
        compiler_params=pltpu.CompilerParams(needs_layout_passes=False),
        scratch_types=[
            pltpu.VMEM((seq, bpw), jnp.int32),
            pltpu.VMEM((seq, bpw), jnp.int32),
            [pltpu.VMEM((2 * bpw, DIM_TOK), jnp.float32) for _ in range(2)],
            [pltpu.VMEM((DIM_OUT, bpw), jnp.float32) for _ in range(2)],
            [pltpu.SemaphoreType.DMA for _ in range(2)],
            [pltpu.SemaphoreType.DMA for _ in range(2)],
        ],
    )
    def embed(tok_hbm, pos_hbm, wt_hbm, wp_hbm, out_hbm,
              tok_idx, pos_idx, gbufs, tbufs, gsems, osems):
        wid = lax.axis_index("s") * _NC + lax.axis_index("c")
        b0 = wid * bpw

        pltpu.sync_copy(tok_hbm.at[:, pl.ds(b0, bpw)], tok_idx)
        pltpu.sync_copy(pos_hbm.at[:, pl.ds(b0, bpw)], pos_idx)

        def issue_gather(g, k):
            pltpu.async_copy(wp_hbm.at[pos_idx.at[g]],
                             gbufs[k].at[pl.ds(0, bpw), :], gsems[k])
            pltpu.async_copy(wt_hbm.at[tok_idx.at[g]],
                             gbufs[k].at[pl.ds(bpw, bpw), :], gsems[k])

        def drain_gather(g, k):
            pltpu.make_async_copy(wp_hbm.at[pos_idx.at[g]],
                                  gbufs[k].at[pl.ds(0, bpw), :],
                                  gsems[k]).wait()
            pltpu.make_async_copy(wt_hbm.at[tok_idx.at[g]],
                                  gbufs[k].at[pl.ds(bpw, bpw), :],
                                  gsems[k]).wait()

        def transpose(k):
            gbuf = gbufs[k]
            tbuf = tbufs[k]
            iota = lax.iota(jnp.int32, _L)

            def row_pos(f, carry):
                colv = jnp.full((_L,), f, jnp.int32)
                for m in range(bpw // _L):
                    v = plsc.load_gather(gbuf, [iota + (_L * m), colv])
                    tbuf[f, pl.ds(_L * m, _L)] = v
                return carry

            def row_tok(f, carry):
                colv = jnp.full((_L,), f, jnp.int32)
                for m in range(bpw // _L):
                    v = plsc.load_gather(gbuf, [iota + (bpw + _L * m), colv])
                    tbuf[DIM_POS + f, pl.ds(_L * m, _L)] = v
                return carry

            lax.fori_loop(0, DIM_POS, row_pos, 0)
            lax.fori_loop(0, DIM_TOK, row_tok, 0)

        def issue_out(g, k):
            pltpu.async_copy(tbufs[k], out_hbm.at[g, :, pl.ds(b0, bpw)],
                             osems[k])

        def drain_out(k):
            pltpu.make_async_copy(tbufs[k], out_hbm.at[0, :, pl.ds(b0, bpw)],
                                  osems[k]).wait()

        def body(j, carry):
            for kk in range(2):
                i = 2 * j + kk

                @pl.when((i >= 2) & (i < seq + 2))
                def _(i=i, kk=kk):
                    drain_gather(i - 2, kk)

                @pl.when((i >= 4) & (i < seq + 4))
                def _(i=i, kk=kk):
                    drain_out(kk)

                @pl.when((i >= 2) & (i < seq + 2))
                def _(i=i, kk=kk):
                    transpose(kk)
                    issue_out(i - 2, kk)

                @pl.when(i < seq)
                def _(i=i, kk=kk):
                    issue_gather(i, kk)

            return carry

        lax.fori_loop(0, (seq + 4 + 1) // 2, body, 0)

    return embed


def kernel(tokens, pos, W_tokens, W_pos):
    batch, seq = tokens.shape
    tok_t = jnp.transpose(tokens.astype(jnp.int32))
    pos_t = jnp.transpose(pos.astype(jnp.int32))
    wp_pad = jnp.pad(W_pos, ((0, 0), (0, DIM_TOK - DIM_POS)))
    out_p = _make_kernel(batch, seq)(tok_t, pos_t, W_tokens, wp_pad)
    return jnp.transpose(out_p, (2, 0, 1))


# speedup vs baseline: 2.3158x; 2.3158x over previous
import functools
import jax
import jax.numpy as jnp
from jax import lax
from jax.experimental import pallas as pl
from jax.experimental.pallas import tpu as pltpu
from jax.experimental.pallas import tpu_sc as plsc

DIM_POS = 64
DIM_TOK = 128
DIM_OUT = DIM_POS + DIM_TOK

_NC = 2
_NS = 16
_NW = _NC * _NS
_NSLOT = 4
_LANES = 16


def _make_kernel(batch, seq):
    assert batch % _NW == 0
    bpw = batch // _NW
    n_iter = bpw + _NSLOT
    mesh = plsc.VectorSubcoreMesh(core_axis_name="c", subcore_axis_name="s")

    @functools.partial(
        pl.kernel,
        out_type=jax.ShapeDtypeStruct((batch, seq, DIM_OUT), jnp.float32),
        mesh=mesh,
        scratch_types=[
            pltpu.VMEM((bpw, seq), jnp.int32),
            pltpu.VMEM((bpw, seq), jnp.int32),
            [pltpu.VMEM((seq, DIM_OUT), jnp.float32) for _ in range(_NSLOT)],
            [pltpu.VMEM((seq, DIM_TOK), jnp.float32) for _ in range(_NSLOT)],
            [pltpu.SemaphoreType.DMA for _ in range(_NSLOT)],
            [pltpu.SemaphoreType.DMA for _ in range(_NSLOT)],
        ],
    )
    def embed(tok_hbm, pos_hbm, wt_hbm, wp_hbm, out_hbm,
              tok_idx, pos_idx, comb_bufs, tok_bufs, gsems, osems):
        wid = lax.axis_index("s") * _NC + lax.axis_index("c")
        b0 = wid * bpw

        pltpu.sync_copy(tok_hbm.at[pl.ds(b0, bpw)], tok_idx)
        pltpu.sync_copy(pos_hbm.at[pl.ds(b0, bpw)], pos_idx)

        def issue_gather(g, s):
            pltpu.async_copy(wp_hbm.at[pos_idx.at[g]],
                             comb_bufs[s].at[:, pl.ds(0, DIM_TOK)], gsems[s])
            pltpu.async_copy(wt_hbm.at[tok_idx.at[g]], tok_bufs[s], gsems[s])

        def drain_gather(g, s):
            pltpu.make_async_copy(wp_hbm.at[pos_idx.at[g]],
                                  comb_bufs[s].at[:, pl.ds(0, DIM_TOK)],
                                  gsems[s]).wait()
            pltpu.make_async_copy(wt_hbm.at[tok_idx.at[g]], tok_bufs[s],
                                  gsems[s]).wait()

        def fixup(s):
            comb = comb_bufs[s]
            tokb = tok_bufs[s]

            def row(r, carry):
                for c in range(DIM_TOK // _LANES):
                    comb[r, pl.ds(DIM_POS + c * _LANES, _LANES)] = (
                        tokb[r, pl.ds(c * _LANES, _LANES)])
                return carry

            lax.fori_loop(0, seq, row, 0)

        def issue_out(g, s):
            pltpu.async_copy(comb_bufs[s], out_hbm.at[b0 + g], osems[s])

        def drain_out(s):
            pltpu.make_async_copy(comb_bufs[s], out_hbm.at[0], osems[s]).wait()

        def body(j, carry):
            for k in range(_NSLOT):
                i = j * _NSLOT + k

                @pl.when((i >= _NSLOT) & (i < bpw + _NSLOT))
                def _(i=i, k=k):
                    drain_out(k)

                @pl.when(i < bpw)
                def _(i=i, k=k):
                    issue_gather(i, k)

                @pl.when((i >= 2) & (i < bpw + 2))
                def _(i=i, k=k):
                    s = (k + _NSLOT - 2) % _NSLOT
                    drain_gather(i - 2, s)
                    fixup(s)
                    issue_out(i - 2, s)

            return carry

        lax.fori_loop(0, (n_iter + _NSLOT - 1) // _NSLOT, body, 0)

    return embed


def kernel(tokens, pos, W_tokens, W_pos):
    batch, seq = tokens.shape
    wp_pad = jnp.pad(W_pos, ((0, 0), (0, DIM_TOK - DIM_POS)))
    return _make_kernel(batch, seq)(
        tokens.astype(jnp.int32), pos.astype(jnp.int32), W_tokens, wp_pad)
